# baseline (speedup 1.0000x reference)
import functools

import jax
import jax.numpy as jnp
from jax import lax
from jax.experimental import pallas as pl
from jax.experimental.pallas import tpu as pltpu
from jax.experimental.pallas import tpu_sc as plsc

VOCAB = 1_000_000
MAXLEN = 200
EMBED = 64
BATCH = 4096

_INFO = plsc.get_sparse_core_info()
NC, NS = _INFO.num_cores, _INFO.num_subcores
NW = NC * NS
HALF = 100
NHALF = BATCH * MAXLEN // HALF
H_PER_W = NHALF // NW
K = 4
G = H_PER_W // K
CHUNK = K * HALF


def _body(x_hbm, tok_hbm, pos_hbm, out_hbm, idx_v, pos_v, buf_v, gsem):
    wid = lax.axis_index("s") * NC + lax.axis_index("c")

    pltpu.sync_copy(pos_hbm, pos_v.at[pl.ds(0, MAXLEN)])
    pltpu.sync_copy(pos_hbm, pos_v.at[pl.ds(MAXLEN, MAXLEN)])

    def chunk(g, carry):
        hs0 = wid * H_PER_W + g * K
        row0 = hs0 * HALF
        pltpu.sync_copy(x_hbm.at[pl.ds(hs0, K)], idx_v)
        copies = [
            pltpu.async_copy(tok_hbm.at[idx_v.at[k]],
                             buf_v.at[pl.ds(k * HALF, HALF)], gsem)
            for k in range(K)
        ]
        for c in copies:
            c.wait()

        def add_rows(i, c2):
            for rr in range(4):
                r = i * 4 + rr
                for j in range(EMBED // 16):
                    plsc.addupdate(buf_v.at[r, pl.ds(j * 16, 16)],
                                   pos_v[r, pl.ds(j * 16, 16)])
            return c2

        lax.fori_loop(0, CHUNK // 4, add_rows, 0)
        pltpu.sync_copy(buf_v, out_hbm.at[pl.ds(row0, CHUNK)])
        return carry

    lax.fori_loop(0, G, chunk, 0)


def kernel(x, token_table, pos_table):
    x2 = x.reshape(NHALF, HALF).astype(jnp.int32)
    mesh = plsc.VectorSubcoreMesh(core_axis_name="c", subcore_axis_name="s")
    run = functools.partial(
        pl.kernel,
        out_type=jax.ShapeDtypeStruct((BATCH * MAXLEN, EMBED), jnp.float32),
        mesh=mesh,
        compiler_params=pltpu.CompilerParams(use_tc_tiling_on_sc=False),
        scratch_types=[
            pltpu.VMEM((K, HALF), jnp.int32),
            pltpu.VMEM((CHUNK, EMBED), jnp.float32),
            pltpu.VMEM((CHUNK, EMBED), jnp.float32),
            pltpu.SemaphoreType.DMA,
        ],
    )(_body)
    out = run(x2, token_table, pos_table)
    return out.reshape(BATCH, MAXLEN, EMBED)


# speedup vs baseline: 1.0800x; 1.0800x over previous
import functools

import jax
import jax.numpy as jnp
from jax import lax
from jax.experimental import pallas as pl
from jax.experimental.pallas import tpu as pltpu
from jax.experimental.pallas import tpu_sc as plsc

VOCAB = 1_000_000
MAXLEN = 200
EMBED = 64
BATCH = 4096

_INFO = plsc.get_sparse_core_info()
NC, NS = _INFO.num_cores, _INFO.num_subcores
NW = NC * NS
HALF = 100
NHALF = BATCH * MAXLEN // HALF
H_PER_W = NHALF // NW
K = 4
G = H_PER_W // K
CHUNK = K * HALF


def _body(x_hbm, tok_hbm, pos_hbm, out_hbm, idx_v, pos_v, buf_v, gsem, osem):
    wid = lax.axis_index("s") * NC + lax.axis_index("c")

    pltpu.sync_copy(pos_hbm, pos_v.at[pl.ds(0, MAXLEN)])
    pltpu.sync_copy(pos_hbm, pos_v.at[pl.ds(MAXLEN, MAXLEN)])

    def fire(g, s):
        hs0 = wid * H_PER_W + g * K
        pltpu.sync_copy(x_hbm.at[pl.ds(hs0, K)], idx_v.at[s])
        for k in range(K):
            pltpu.async_copy(tok_hbm.at[idx_v.at[s, k]],
                             buf_v.at[s, pl.ds(k * HALF, HALF)], gsem.at[s])

    def drain_gathers(s):
        for k in range(K):
            pltpu.make_async_copy(tok_hbm.at[idx_v.at[s, k]],
                                  buf_v.at[s, pl.ds(k * HALF, HALF)],
                                  gsem.at[s]).wait()

    def add_pos(s):
        def add_rows(i, c2):
            for rr in range(4):
                r = i * 4 + rr
                for j in range(EMBED // 16):
                    plsc.addupdate(buf_v.at[s, r, pl.ds(j * 16, 16)],
                                   pos_v[r, pl.ds(j * 16, 16)])
            return c2
        lax.fori_loop(0, CHUNK // 4, add_rows, 0, unroll=4)

    def out_copy(g, s):
        row0 = (wid * H_PER_W + g * K) * HALF
        pltpu.async_copy(buf_v.at[s], out_hbm.at[pl.ds(row0, CHUNK)], osem.at[s])

    def wait_out(s):
        pltpu.make_async_copy(buf_v.at[s],
                              out_hbm.at[pl.ds(0, CHUNK)], osem.at[s]).wait()

    def step(g, carry):
        for s in (0, 1):
            gg = g * 2 + s
            o = 1 - s

            @pl.when(gg >= 2)
            def _():
                wait_out(s)

            fire(gg, s)

            @pl.when(gg >= 1)
            def _():
                drain_gathers(o)
                add_pos(o)
                out_copy(gg - 1, o)
        return carry

    lax.fori_loop(0, G // 2, step, 0)

    drain_gathers(1)
    add_pos(1)
    out_copy(G - 1, 1)
    wait_out(0)
    wait_out(1)


def kernel(x, token_table, pos_table):
    x2 = x.reshape(NHALF, HALF).astype(jnp.int32)
    mesh = plsc.VectorSubcoreMesh(core_axis_name="c", subcore_axis_name="s")
    run = functools.partial(
        pl.kernel,
        out_type=jax.ShapeDtypeStruct((BATCH * MAXLEN, EMBED), jnp.float32),
        mesh=mesh,
        compiler_params=pltpu.CompilerParams(use_tc_tiling_on_sc=False),
        scratch_types=[
            pltpu.VMEM((2, K, HALF), jnp.int32),
            pltpu.VMEM((CHUNK, EMBED), jnp.float32),
            pltpu.VMEM((2, CHUNK, EMBED), jnp.float32),
            pltpu.SemaphoreType.DMA((2,)),
            pltpu.SemaphoreType.DMA((2,)),
        ],
    )(_body)
    out = run(x2, token_table, pos_table)
    return out.reshape(BATCH, MAXLEN, EMBED)
